# baseline (speedup 1.0000x reference)
import functools

import jax
import jax.numpy as jnp
from jax import lax
from jax.experimental import pallas as pl
from jax.experimental.pallas import tpu as pltpu
from jax.experimental.pallas import tpu_sc as plsc

N_ROWS = 1_000_000
N_ATTRS = 4
N_TABLE = 1024
CHUNK = 4_608
N_MAIN = 999_936
N_CHUNKS = N_MAIN // CHUNK
TAIL = N_ROWS - N_MAIN
LANES = 16
NUM_CORES = 2
NUM_SUBCORES = 16
NW = NUM_CORES * NUM_SUBCORES


def _sc_gather_add(hp_t, ids_t, delta):
    mesh = plsc.VectorSubcoreMesh(core_axis_name="c", subcore_axis_name="s")

    @functools.partial(
        pl.kernel,
        out_type=jax.ShapeDtypeStruct((N_ATTRS, N_ROWS), jnp.float32),
        mesh=mesh,
        compiler_params=pltpu.CompilerParams(needs_layout_passes=False),
        scratch_types=[
            pltpu.VMEM((N_TABLE,), jnp.float32),
            pltpu.VMEM((N_ATTRS, CHUNK), jnp.int32),
            pltpu.VMEM((N_ATTRS, CHUNK), jnp.float32),
            pltpu.VMEM((N_ATTRS, CHUNK), jnp.int32),
            pltpu.VMEM((N_ATTRS, CHUNK), jnp.float32),
            pltpu.VMEM((N_ATTRS, CHUNK), jnp.float32),
            pltpu.VMEM((N_ATTRS, CHUNK), jnp.float32),
            pltpu.VMEM((N_ATTRS, TAIL), jnp.int32),
            pltpu.VMEM((N_ATTRS, TAIL), jnp.float32),
            pltpu.SemaphoreType.DMA,
            pltpu.SemaphoreType.DMA,
            pltpu.SemaphoreType.DMA,
            pltpu.SemaphoreType.DMA,
        ],
    )
    def k(hp_hbm, ids_hbm, delta_hbm, out_hbm, delta_v,
          ids_v0, hp_v0, ids_v1, hp_v1, out_v0, out_v1,
          ids_tail_v, hp_tail_v,
          in_sem0, in_sem1, out_sem0, out_sem1):
        wid = lax.axis_index("s") * NUM_CORES + lax.axis_index("c")
        pltpu.sync_copy(delta_hbm, delta_v)
        n_w = (N_CHUNKS - wid + NW - 1) // NW
        bufs = (
            (ids_v0, hp_v0, out_v0, in_sem0, out_sem0),
            (ids_v1, hp_v1, out_v1, in_sem1, out_sem1),
        )

        def c0_of(i):
            return (wid + i * NW) * CHUNK

        def start_in(i, ids_v, hp_v, in_sem):
            c0 = c0_of(i)
            pltpu.make_async_copy(
                ids_hbm.at[:, pl.ds(c0, CHUNK)], ids_v, in_sem
            ).start()
            pltpu.make_async_copy(
                hp_hbm.at[:, pl.ds(c0, CHUNK)], hp_v, in_sem
            ).start()

        def wait_in(ids_v, hp_v, in_sem):
            pltpu.make_async_copy(
                ids_hbm.at[:, pl.ds(0, CHUNK)], ids_v, in_sem
            ).wait()
            pltpu.make_async_copy(
                hp_hbm.at[:, pl.ds(0, CHUNK)], hp_v, in_sem
            ).wait()

        def compute(ids_v, hp_v, out_v):
            @plsc.parallel_loop(0, CHUNK, step=LANES, unroll=8)
            def _body(off):
                for j in range(N_ATTRS):
                    idx = ids_v[j, pl.ds(off, LANES)]
                    g = plsc.load_gather(delta_v, [idx])
                    out_v[j, pl.ds(off, LANES)] = hp_v[j, pl.ds(off, LANES)] + g

        def start_out(i, out_v, out_sem):
            pltpu.make_async_copy(
                out_v, out_hbm.at[:, pl.ds(c0_of(i), CHUNK)], out_sem
            ).start()

        def wait_out(out_v, out_sem):
            pltpu.make_async_copy(
                out_v, out_hbm.at[:, pl.ds(0, CHUNK)], out_sem
            ).wait()

        start_in(0, ids_v0, hp_v0, in_sem0)
        start_in(1, ids_v1, hp_v1, in_sem1)

        def chunk_body(i, carry):
            for b in range(2):
                @pl.when(lax.rem(i, 2) == b)
                def _step():
                    ids_v, hp_v, out_v, in_sem, out_sem = bufs[b]
                    wait_in(ids_v, hp_v, in_sem)

                    @pl.when(i >= 2)
                    def _():
                        wait_out(out_v, out_sem)

                    compute(ids_v, hp_v, out_v)
                    start_out(i, out_v, out_sem)

                    @pl.when(i + 2 < n_w)
                    def _():
                        start_in(i + 2, ids_v, hp_v, in_sem)

            return carry

        lax.fori_loop(0, n_w, chunk_body, 0)
        for b in range(2):
            wait_out(bufs[b][2], bufs[b][4])

        @pl.when(wid == NW - 1)
        def _tail():
            pltpu.sync_copy(ids_hbm.at[:, pl.ds(N_MAIN, TAIL)], ids_tail_v)
            pltpu.sync_copy(hp_hbm.at[:, pl.ds(N_MAIN, TAIL)], hp_tail_v)
            for j in range(N_ATTRS):
                for v in range(TAIL // LANES):
                    off = v * LANES
                    idx = ids_tail_v[j, pl.ds(off, LANES)]
                    g = plsc.load_gather(delta_v, [idx])
                    hp_tail_v[j, pl.ds(off, LANES)] = (
                        hp_tail_v[j, pl.ds(off, LANES)] + g
                    )
            pltpu.sync_copy(hp_tail_v, out_hbm.at[:, pl.ds(N_MAIN, TAIL)])

    return k(hp_t, ids_t, delta)


def kernel(handler_parameters, parameter_ids_map, parameter_delta):
    out_t = _sc_gather_add(
        handler_parameters.T, parameter_ids_map.T, parameter_delta
    )
    return out_t.T


# speedup vs baseline: 1.0012x; 1.0012x over previous
import functools

import jax
import jax.numpy as jnp
from jax import lax
from jax.experimental import pallas as pl
from jax.experimental.pallas import tpu as pltpu
from jax.experimental.pallas import tpu_sc as plsc

N_ROWS = 1_000_000
N_ATTRS = 4
N_TABLE = 1024
CHUNK = 3_584
N_MAIN = 999_936
N_CHUNKS = N_MAIN // CHUNK
TAIL = N_ROWS - N_MAIN
LANES = 16
NUM_CORES = 2
NUM_SUBCORES = 16
NW = NUM_CORES * NUM_SUBCORES


def _sc_gather_add(hp_t, ids_t, delta):
    mesh = plsc.VectorSubcoreMesh(core_axis_name="c", subcore_axis_name="s")

    @functools.partial(
        pl.kernel,
        out_type=jax.ShapeDtypeStruct((N_ATTRS, N_ROWS), jnp.float32),
        mesh=mesh,
        compiler_params=pltpu.CompilerParams(needs_layout_passes=False),
        scratch_types=[
            pltpu.VMEM((N_TABLE,), jnp.float32),
            pltpu.VMEM((N_ATTRS, CHUNK), jnp.int32),
            pltpu.VMEM((N_ATTRS, CHUNK), jnp.float32),
            pltpu.VMEM((N_ATTRS, CHUNK), jnp.int32),
            pltpu.VMEM((N_ATTRS, CHUNK), jnp.float32),
            pltpu.VMEM((N_ATTRS, CHUNK), jnp.float32),
            pltpu.VMEM((N_ATTRS, CHUNK), jnp.float32),
            pltpu.VMEM((N_ATTRS, TAIL), jnp.int32),
            pltpu.VMEM((N_ATTRS, TAIL), jnp.float32),
            pltpu.SemaphoreType.DMA,
            pltpu.SemaphoreType.DMA,
            pltpu.SemaphoreType.DMA,
            pltpu.SemaphoreType.DMA,
        ],
    )
    def k(hp_hbm, ids_hbm, delta_hbm, out_hbm, delta_v,
          ids_v0, hp_v0, ids_v1, hp_v1, out_v0, out_v1,
          ids_tail_v, hp_tail_v,
          in_sem0, in_sem1, out_sem0, out_sem1):
        wid = lax.axis_index("s") * NUM_CORES + lax.axis_index("c")
        pltpu.sync_copy(delta_hbm, delta_v)
        n_w = (N_CHUNKS - wid + NW - 1) // NW
        bufs = (
            (ids_v0, hp_v0, out_v0, in_sem0, out_sem0),
            (ids_v1, hp_v1, out_v1, in_sem1, out_sem1),
        )

        def c0_of(i):
            return (wid + i * NW) * CHUNK

        def start_in(i, ids_v, hp_v, in_sem):
            c0 = c0_of(i)
            pltpu.make_async_copy(
                ids_hbm.at[:, pl.ds(c0, CHUNK)], ids_v, in_sem
            ).start()
            pltpu.make_async_copy(
                hp_hbm.at[:, pl.ds(c0, CHUNK)], hp_v, in_sem
            ).start()

        def wait_in(ids_v, hp_v, in_sem):
            pltpu.make_async_copy(
                ids_hbm.at[:, pl.ds(0, CHUNK)], ids_v, in_sem
            ).wait()
            pltpu.make_async_copy(
                hp_hbm.at[:, pl.ds(0, CHUNK)], hp_v, in_sem
            ).wait()

        def compute(ids_v, hp_v, out_v):
            @plsc.parallel_loop(0, CHUNK, step=LANES, unroll=8)
            def _body(off):
                for j in range(N_ATTRS):
                    idx = ids_v[j, pl.ds(off, LANES)]
                    g = plsc.load_gather(delta_v, [idx])
                    out_v[j, pl.ds(off, LANES)] = hp_v[j, pl.ds(off, LANES)] + g

        def start_out(i, out_v, out_sem):
            pltpu.make_async_copy(
                out_v, out_hbm.at[:, pl.ds(c0_of(i), CHUNK)], out_sem
            ).start()

        def wait_out(out_v, out_sem):
            pltpu.make_async_copy(
                out_v, out_hbm.at[:, pl.ds(0, CHUNK)], out_sem
            ).wait()

        start_in(0, ids_v0, hp_v0, in_sem0)
        start_in(1, ids_v1, hp_v1, in_sem1)

        def chunk_body(i, carry):
            for b in range(2):
                @pl.when(lax.rem(i, 2) == b)
                def _step():
                    ids_v, hp_v, out_v, in_sem, out_sem = bufs[b]
                    wait_in(ids_v, hp_v, in_sem)

                    @pl.when(i >= 2)
                    def _():
                        wait_out(out_v, out_sem)

                    compute(ids_v, hp_v, out_v)
                    start_out(i, out_v, out_sem)

                    @pl.when(i + 2 < n_w)
                    def _():
                        start_in(i + 2, ids_v, hp_v, in_sem)

            return carry

        lax.fori_loop(0, n_w, chunk_body, 0)
        for b in range(2):
            wait_out(bufs[b][2], bufs[b][4])

        @pl.when(wid == NW - 1)
        def _tail():
            pltpu.sync_copy(ids_hbm.at[:, pl.ds(N_MAIN, TAIL)], ids_tail_v)
            pltpu.sync_copy(hp_hbm.at[:, pl.ds(N_MAIN, TAIL)], hp_tail_v)
            for j in range(N_ATTRS):
                for v in range(TAIL // LANES):
                    off = v * LANES
                    idx = ids_tail_v[j, pl.ds(off, LANES)]
                    g = plsc.load_gather(delta_v, [idx])
                    hp_tail_v[j, pl.ds(off, LANES)] = (
                        hp_tail_v[j, pl.ds(off, LANES)] + g
                    )
            pltpu.sync_copy(hp_tail_v, out_hbm.at[:, pl.ds(N_MAIN, TAIL)])

    return k(hp_t, ids_t, delta)


def kernel(handler_parameters, parameter_ids_map, parameter_delta):
    out_t = _sc_gather_add(
        handler_parameters.T, parameter_ids_map.T, parameter_delta
    )
    return out_t.T


# speedup vs baseline: 1.0246x; 1.0234x over previous
import functools

import jax
import jax.numpy as jnp
from jax import lax
from jax.experimental import pallas as pl
from jax.experimental.pallas import tpu as pltpu
from jax.experimental.pallas import tpu_sc as plsc

N_ROWS = 1_000_000
N_ATTRS = 4
N_TABLE = 1024
CHUNK = 3_584
N_MAIN = 999_936
N_CHUNKS = N_MAIN // CHUNK
TAIL = N_ROWS - N_MAIN
LANES = 16
NUM_CORES = 2
NUM_SUBCORES = 16
NW = NUM_CORES * NUM_SUBCORES


def _sc_gather_add(hp_t, ids_t, delta):
    mesh = plsc.VectorSubcoreMesh(core_axis_name="c", subcore_axis_name="s")

    @functools.partial(
        pl.kernel,
        out_type=jax.ShapeDtypeStruct((N_ATTRS, N_ROWS), jnp.float32),
        mesh=mesh,
        compiler_params=pltpu.CompilerParams(needs_layout_passes=False),
        scratch_types=[
            pltpu.VMEM((N_TABLE,), jnp.float32),
            pltpu.VMEM((N_ATTRS, CHUNK), jnp.int32),
            pltpu.VMEM((N_ATTRS, CHUNK), jnp.float32),
            pltpu.VMEM((N_ATTRS, CHUNK), jnp.int32),
            pltpu.VMEM((N_ATTRS, CHUNK), jnp.float32),
            pltpu.VMEM((N_ATTRS, CHUNK), jnp.float32),
            pltpu.VMEM((N_ATTRS, CHUNK), jnp.float32),
            pltpu.VMEM((N_ATTRS, TAIL), jnp.int32),
            pltpu.VMEM((N_ATTRS, TAIL), jnp.float32),
            pltpu.SemaphoreType.DMA,
            pltpu.SemaphoreType.DMA,
            pltpu.SemaphoreType.DMA,
            pltpu.SemaphoreType.DMA,
            pltpu.SemaphoreType.DMA,
        ],
    )
    def k(hp_hbm, ids_hbm, delta_hbm, out_hbm, delta_v,
          ids_v0, hp_v0, ids_v1, hp_v1, out_v0, out_v1,
          ids_tail_v, hp_tail_v,
          in_sem0, in_sem1, out_sem0, out_sem1, delta_sem):
        wid = lax.axis_index("s") * NUM_CORES + lax.axis_index("c")
        pltpu.make_async_copy(delta_hbm, delta_v, delta_sem).start()
        n_w = (N_CHUNKS - wid + NW - 1) // NW
        bufs = (
            (ids_v0, hp_v0, out_v0, in_sem0, out_sem0),
            (ids_v1, hp_v1, out_v1, in_sem1, out_sem1),
        )

        def c0_of(i):
            return (wid + i * NW) * CHUNK

        def start_in(i, ids_v, hp_v, in_sem):
            c0 = c0_of(i)
            pltpu.make_async_copy(
                ids_hbm.at[:, pl.ds(c0, CHUNK)], ids_v, in_sem
            ).start()
            pltpu.make_async_copy(
                hp_hbm.at[:, pl.ds(c0, CHUNK)], hp_v, in_sem
            ).start()

        def wait_in(ids_v, hp_v, in_sem):
            pltpu.make_async_copy(
                ids_hbm.at[:, pl.ds(0, CHUNK)], ids_v, in_sem
            ).wait()
            pltpu.make_async_copy(
                hp_hbm.at[:, pl.ds(0, CHUNK)], hp_v, in_sem
            ).wait()

        def compute(ids_v, hp_v, out_v):
            @plsc.parallel_loop(0, CHUNK, step=LANES, unroll=8)
            def _body(off):
                for j in range(N_ATTRS):
                    idx = ids_v[j, pl.ds(off, LANES)]
                    g = plsc.load_gather(delta_v, [idx])
                    out_v[j, pl.ds(off, LANES)] = hp_v[j, pl.ds(off, LANES)] + g

        def start_out(i, out_v, out_sem):
            pltpu.make_async_copy(
                out_v, out_hbm.at[:, pl.ds(c0_of(i), CHUNK)], out_sem
            ).start()

        def wait_out(out_v, out_sem):
            pltpu.make_async_copy(
                out_v, out_hbm.at[:, pl.ds(0, CHUNK)], out_sem
            ).wait()

        start_in(0, ids_v0, hp_v0, in_sem0)
        start_in(1, ids_v1, hp_v1, in_sem1)
        pltpu.make_async_copy(delta_hbm, delta_v, delta_sem).wait()

        def chunk_body(i, carry):
            for b in range(2):
                @pl.when(lax.rem(i, 2) == b)
                def _step():
                    ids_v, hp_v, out_v, in_sem, out_sem = bufs[b]
                    wait_in(ids_v, hp_v, in_sem)

                    @pl.when(i >= 2)
                    def _():
                        wait_out(out_v, out_sem)

                    compute(ids_v, hp_v, out_v)
                    start_out(i, out_v, out_sem)

                    @pl.when(i + 2 < n_w)
                    def _():
                        start_in(i + 2, ids_v, hp_v, in_sem)

            return carry

        lax.fori_loop(0, n_w, chunk_body, 0)
        for b in range(2):
            wait_out(bufs[b][2], bufs[b][4])

        @pl.when(wid == NW - 1)
        def _tail():
            pltpu.sync_copy(ids_hbm.at[:, pl.ds(N_MAIN, TAIL)], ids_tail_v)
            pltpu.sync_copy(hp_hbm.at[:, pl.ds(N_MAIN, TAIL)], hp_tail_v)
            for j in range(N_ATTRS):
                for v in range(TAIL // LANES):
                    off = v * LANES
                    idx = ids_tail_v[j, pl.ds(off, LANES)]
                    g = plsc.load_gather(delta_v, [idx])
                    hp_tail_v[j, pl.ds(off, LANES)] = (
                        hp_tail_v[j, pl.ds(off, LANES)] + g
                    )
            pltpu.sync_copy(hp_tail_v, out_hbm.at[:, pl.ds(N_MAIN, TAIL)])

    return k(hp_t, ids_t, delta)


def kernel(handler_parameters, parameter_ids_map, parameter_delta):
    out_t = _sc_gather_add(
        handler_parameters.T, parameter_ids_map.T, parameter_delta
    )
    return out_t.T
